# baseline (speedup 1.0000x reference)
import jax
import jax.numpy as jnp
import numpy as np
from jax import lax
from jax.experimental import pallas as pl
from jax.experimental.pallas import tpu as pltpu
from jax.experimental.pallas import tpu_sc as plsc

N = 10000
E = 320000
D_IN = 128
D_OUT = 40
DP = 48
NC, NS, L = 2, 16, 16
NW = NC * NS
CHUNK = 128
NB = 4
CPW = 80
EPW = CPW * CHUNK
EP = EPW * NW
NP = 10240
RPT = NP // NS


def _mlp_body(x_ref, w0_ref, w1_ref, o_ref):
    h = jnp.maximum(
        jnp.dot(x_ref[...], w0_ref[...], preferred_element_type=jnp.float32), 0.0)
    o_ref[pl.ds(0, N), :] = jnp.dot(h, w1_ref[...],
                                    preferred_element_type=jnp.float32)


_mlp = pl.pallas_call(
    _mlp_body,
    out_shape=jax.ShapeDtypeStruct((NP, DP), jnp.float32),
)


def _mid_body(p_ref, o_ref):
    o_ref[...] = p_ref[0] + p_ref[1]


_mid = pl.pallas_call(
    _mid_body,
    out_shape=jax.ShapeDtypeStruct((NP, DP), jnp.float32),
)


def _final_body(p_ref, o_ref):
    o_ref[...] = p_ref[0, :N, :D_OUT] + p_ref[1, :N, :D_OUT]


_final = pl.pallas_call(
    _final_body,
    out_shape=jax.ShapeDtypeStruct((N, D_OUT), jnp.float32),
)


_GDN = lax.GatherDimensionNumbers(
    offset_dims=(), collapsed_slice_dims=(0,), start_index_map=(0,))

def _spmm_body(h_hbm, src_hbm, dst_hbm, vals_hbm, out_hbm,
               src_v, dst_v, vals_v, rows_v, drain_v, acc,
               gsem, ssem):
    cid = lax.axis_index("c")
    sid = lax.axis_index("s")
    wid = sid * NC + cid

    def zrow(r, carry):
        for k in range(DP // L):
            drain_v[r, pl.ds(k * L, L)] = jnp.zeros((L,), jnp.float32)
        return carry

    lax.fori_loop(0, RPT, zrow, 0)
    pltpu.sync_copy(drain_v, acc.at[pl.ds(sid * RPT, RPT)])

    pltpu.sync_copy(src_hbm.at[wid], src_v)
    pltpu.sync_copy(dst_hbm.at[wid], dst_v)
    pltpu.sync_copy(vals_hbm.at[wid], vals_v)

    plsc.subcore_barrier()

    for b in range(NB):
        pltpu.async_copy(h_hbm.at[src_v.at[b]], rows_v.at[b], gsem[b])

    def outer(g, carry):
        for b in range(NB):
            j = g * NB + b
            pltpu.make_async_copy(
                h_hbm.at[src_v.at[j]], rows_v.at[b], gsem[b]).wait()

            @plsc.parallel_loop(0, CHUNK, L, unroll=2)
            def _scale(c0):
                v16 = vals_v[pl.ds(j * CHUNK + c0, L)]
                for r in range(L):
                    v = lax.gather(
                        v16, jnp.full((L, 1), r, jnp.int32), _GDN,
                        slice_sizes=(1,),
                        mode=lax.GatherScatterMode.PROMISE_IN_BOUNDS)
                    for k in range(DP // L):
                        seg = rows_v[b, c0 + r, pl.ds(k * L, L)]
                        rows_v[b, c0 + r, pl.ds(k * L, L)] = seg * v

            pltpu.async_copy(rows_v.at[b], acc.at[dst_v.at[j]], ssem[b],
                             add=True)

            b2 = (b - 1) % NB
            j2 = j - 1
            jn = j2 + NB

            @pl.when(j2 >= 0)
            def _():
                pltpu.make_async_copy(
                    rows_v.at[b2], acc.at[dst_v.at[j2]], ssem[b2]).wait()

                @pl.when(jn < CPW)
                def _():
                    pltpu.async_copy(
                        h_hbm.at[src_v.at[jn]], rows_v.at[b2], gsem[b2])
        return carry

    lax.fori_loop(0, CPW // NB, outer, 0)

    pltpu.make_async_copy(
        rows_v.at[NB - 1], acc.at[dst_v.at[CPW - 1]], ssem[NB - 1]).wait()

    plsc.subcore_barrier()

    pltpu.sync_copy(acc.at[pl.ds(sid * RPT, RPT)], drain_v)
    pltpu.sync_copy(drain_v, out_hbm.at[cid].at[pl.ds(sid * RPT, RPT)])


_spmm = pl.kernel(
    _spmm_body,
    out_type=jax.ShapeDtypeStruct((NC, NP, DP), jnp.float32),
    mesh=plsc.VectorSubcoreMesh(core_axis_name="c", subcore_axis_name="s"),
    compiler_params=pltpu.CompilerParams(needs_layout_passes=False,
                                         use_tc_tiling_on_sc=False),
    scratch_types=[
        pltpu.VMEM((CPW, CHUNK), jnp.int32),
        pltpu.VMEM((CPW, CHUNK), jnp.int32),
        pltpu.VMEM((EPW,), jnp.float32),
        pltpu.VMEM((NB, CHUNK, DP), jnp.float32),
        pltpu.VMEM((RPT, DP), jnp.float32),
        pltpu.VMEM_SHARED((NP, DP), jnp.float32),
        [pltpu.SemaphoreType.DMA] * NB,
        [pltpu.SemaphoreType.DMA] * NB,
    ],
)


def kernel(features, edge_index, edge_vals, W0, W1):
    w1p = jnp.pad(W1, ((0, 0), (0, DP - D_OUT)))
    h = _mlp(features, W0, w1p)

    pad = EP - E
    src = jnp.concatenate([edge_index[0], jnp.zeros((pad,), jnp.int32)])
    dst = jnp.concatenate([edge_index[1], jnp.zeros((pad,), jnp.int32)])
    vals = jnp.concatenate([edge_vals, jnp.zeros((pad,), jnp.float32)])
    src3 = src.reshape(NW, CPW, CHUNK)
    dst3 = dst.reshape(NW, CPW, CHUNK)
    vals2 = vals.reshape(NW, EPW)

    part = _spmm(h, src3, dst3, vals2)
    h = _mid(part)
    part = _spmm(h, src3, dst3, vals2)
    return _final(part)


# speedup vs baseline: 1.4632x; 1.4632x over previous
import jax
import jax.numpy as jnp
import numpy as np
from jax import lax
from jax.experimental import pallas as pl
from jax.experimental.pallas import tpu as pltpu
from jax.experimental.pallas import tpu_sc as plsc

N = 10000
E = 320000
D_IN = 128
D_OUT = 40
DP = 48
NC, NS, L = 2, 16, 16
NW = NC * NS
CHUNK = 128
NB = 4
CPW = 80
EPW = CPW * CHUNK
EP = EPW * NW
NP = 10240
RPT = NP // NS


def _mlp_body(x_ref, w0_ref, w1_ref, o_ref):
    h = jnp.maximum(
        jnp.dot(x_ref[...], w0_ref[...], preferred_element_type=jnp.float32), 0.0)
    o_ref[pl.ds(0, N), :] = jnp.dot(h, w1_ref[...],
                                    preferred_element_type=jnp.float32)


_mlp = pl.pallas_call(
    _mlp_body,
    out_shape=jax.ShapeDtypeStruct((NP, DP), jnp.float32),
)


def _mid_body(p_ref, o_ref):
    o_ref[...] = p_ref[0] + p_ref[1]


_mid = pl.pallas_call(
    _mid_body,
    out_shape=jax.ShapeDtypeStruct((NP, DP), jnp.float32),
)


def _final_body(p_ref, o_ref):
    o_ref[...] = p_ref[0, :N, :D_OUT] + p_ref[1, :N, :D_OUT]


_final = pl.pallas_call(
    _final_body,
    out_shape=jax.ShapeDtypeStruct((N, D_OUT), jnp.float32),
)


_GDN = lax.GatherDimensionNumbers(
    offset_dims=(), collapsed_slice_dims=(0,), start_index_map=(0,))

def _spmm_body(h_hbm, src_hbm, dst_hbm, vals_hbm, out_hbm,
               src_v, dst_v, vals_v, rows_v, drain_v, acc,
               gsem, ssem):
    cid = lax.axis_index("c")
    sid = lax.axis_index("s")
    wid = sid * NC + cid

    def zrow(r, carry):
        for k in range(DP // L):
            drain_v[r, pl.ds(k * L, L)] = jnp.zeros((L,), jnp.float32)
        return carry

    lax.fori_loop(0, RPT, zrow, 0)
    pltpu.sync_copy(drain_v, acc.at[pl.ds(sid * RPT, RPT)])

    pltpu.sync_copy(src_hbm.at[wid], src_v)
    pltpu.sync_copy(dst_hbm.at[wid], dst_v)
    pltpu.sync_copy(vals_hbm.at[wid], vals_v)

    plsc.subcore_barrier()

    for b in range(NB):
        pltpu.async_copy(h_hbm.at[pl.ds(0, CHUNK)], rows_v.at[b], gsem[b])

    def outer(g, carry):
        for b in range(NB):
            j = g * NB + b
            pltpu.make_async_copy(
                h_hbm.at[pl.ds(0, CHUNK)], rows_v.at[b], gsem[b]).wait()

            @plsc.parallel_loop(0, 0, L, unroll=2)
            def _scale(c0):
                v16 = vals_v[pl.ds(j * CHUNK + c0, L)]
                for r in range(L):
                    v = lax.gather(
                        v16, jnp.full((L, 1), r, jnp.int32), _GDN,
                        slice_sizes=(1,),
                        mode=lax.GatherScatterMode.PROMISE_IN_BOUNDS)
                    for k in range(DP // L):
                        seg = rows_v[b, c0 + r, pl.ds(k * L, L)]
                        rows_v[b, c0 + r, pl.ds(k * L, L)] = seg * v

            pltpu.async_copy(rows_v.at[b], acc.at[pl.ds(0, CHUNK)], ssem[b])

            b2 = (b - 1) % NB
            j2 = j - 1
            jn = j2 + NB

            @pl.when(j2 >= 0)
            def _():
                pltpu.make_async_copy(
                    rows_v.at[b2], acc.at[pl.ds(0, CHUNK)], ssem[b2]).wait()

                @pl.when(jn < CPW)
                def _():
                    pltpu.async_copy(
                        h_hbm.at[pl.ds(0, CHUNK)], rows_v.at[b2], gsem[b2])
        return carry

    lax.fori_loop(0, CPW // NB, outer, 0)

    pltpu.make_async_copy(
        rows_v.at[NB - 1], acc.at[pl.ds(0, CHUNK)], ssem[NB - 1]).wait()

    plsc.subcore_barrier()

    pltpu.sync_copy(acc.at[pl.ds(sid * RPT, RPT)], drain_v)
    pltpu.sync_copy(drain_v, out_hbm.at[cid].at[pl.ds(sid * RPT, RPT)])


_spmm = pl.kernel(
    _spmm_body,
    out_type=jax.ShapeDtypeStruct((NC, NP, DP), jnp.float32),
    mesh=plsc.VectorSubcoreMesh(core_axis_name="c", subcore_axis_name="s"),
    compiler_params=pltpu.CompilerParams(needs_layout_passes=False,
                                         use_tc_tiling_on_sc=False),
    scratch_types=[
        pltpu.VMEM((CPW, CHUNK), jnp.int32),
        pltpu.VMEM((CPW, CHUNK), jnp.int32),
        pltpu.VMEM((EPW,), jnp.float32),
        pltpu.VMEM((NB, CHUNK, DP), jnp.float32),
        pltpu.VMEM((RPT, DP), jnp.float32),
        pltpu.VMEM_SHARED((NP, DP), jnp.float32),
        [pltpu.SemaphoreType.DMA] * NB,
        [pltpu.SemaphoreType.DMA] * NB,
    ],
)


def kernel(features, edge_index, edge_vals, W0, W1):
    w1p = jnp.pad(W1, ((0, 0), (0, DP - D_OUT)))
    h = _mlp(features, W0, w1p)

    pad = EP - E
    src = jnp.concatenate([edge_index[0], jnp.zeros((pad,), jnp.int32)])
    dst = jnp.concatenate([edge_index[1], jnp.zeros((pad,), jnp.int32)])
    vals = jnp.concatenate([edge_vals, jnp.zeros((pad,), jnp.float32)])
    src3 = src.reshape(NW, CPW, CHUNK)
    dst3 = dst.reshape(NW, CPW, CHUNK)
    vals2 = vals.reshape(NW, EPW)

    part = _spmm(h, src3, dst3, vals2)
    h = _mid(part)
    part = _spmm(h, src3, dst3, vals2)
    return _final(part)


# speedup vs baseline: 2.1156x; 1.4459x over previous
import jax
import jax.numpy as jnp
from jax import lax
from jax.experimental import pallas as pl
from jax.experimental.pallas import tpu as pltpu
from jax.experimental.pallas import tpu_sc as plsc

N = 10000
E = 320000
D_IN = 128
D_OUT = 40
DP = 48
NC, NS, L = 2, 16, 16
NW = NC * NS
CHUNK = 128
NB = 4
CPW = 80
EPW = CPW * CHUNK
EP = EPW * NW
NP = 10240
RPT = NP // NS
NBLK = RPT // CHUNK


def _mlp_body(x_ref, w0_ref, w1_ref, o_ref):
    h = jnp.maximum(
        jnp.dot(x_ref[...], w0_ref[...], preferred_element_type=jnp.float32), 0.0)
    o_ref[0, pl.ds(0, N), :] = jnp.dot(h, w1_ref[...],
                                       preferred_element_type=jnp.float32)


_mlp = pl.pallas_call(
    _mlp_body,
    out_shape=jax.ShapeDtypeStruct((1, NP, DP), jnp.float32),
)


def _final_body(p_ref, o_ref):
    o_ref[...] = p_ref[0, :N, :D_OUT] + p_ref[1, :N, :D_OUT]


_final = pl.pallas_call(
    _final_body,
    out_shape=jax.ShapeDtypeStruct((N, D_OUT), jnp.float32),
)


_GDN = lax.GatherDimensionNumbers(
    offset_dims=(), collapsed_slice_dims=(0,), start_index_map=(0,))


def _make_spmm(n_parts):
    def body(parts_hbm, src_hbm, dst_hbm, vals_hbm, out_hbm,
             src_v, dst_v, vals_v, rows_v, acc, h_s, gsem, ssem):
        cid = lax.axis_index("c")
        sid = lax.axis_index("s")
        wid = sid * NC + cid
        base = sid * RPT

        def zrow(r, carry):
            for k in range(DP // L):
                rows_v[0, r, pl.ds(k * L, L)] = jnp.zeros((L,), jnp.float32)
            return carry

        lax.fori_loop(0, CHUNK, zrow, 0)
        for blk in range(NBLK):
            pltpu.sync_copy(rows_v.at[0],
                            acc.at[pl.ds(base + blk * CHUNK, CHUNK)])

        for blk in range(NBLK):
            sl = pl.ds(base + blk * CHUNK, CHUNK)
            pltpu.sync_copy(parts_hbm.at[0].at[sl], rows_v.at[1])
            if n_parts == 2:
                pltpu.sync_copy(parts_hbm.at[1].at[sl], rows_v.at[2])

                def arow(r, carry):
                    for k in range(DP // L):
                        s = pl.ds(k * L, L)
                        rows_v[1, r, s] = rows_v[1, r, s] + rows_v[2, r, s]
                    return carry

                lax.fori_loop(0, CHUNK, arow, 0)
            pltpu.sync_copy(rows_v.at[1], h_s.at[sl])

        pltpu.sync_copy(src_hbm.at[wid], src_v)
        pltpu.sync_copy(dst_hbm.at[wid], dst_v)
        pltpu.sync_copy(vals_hbm.at[wid], vals_v)

        plsc.subcore_barrier()

        for b in range(NB):
            pltpu.async_copy(h_s.at[src_v.at[b]], rows_v.at[b], gsem[b])

        def outer(g, carry):
            for b in range(NB):
                j = g * NB + b
                pltpu.make_async_copy(
                    h_s.at[src_v.at[j]], rows_v.at[b], gsem[b]).wait()

                @plsc.parallel_loop(0, CHUNK, L, unroll=2)
                def _scale(c0):
                    v16 = vals_v[pl.ds(j * CHUNK + c0, L)]
                    for r in range(L):
                        v = lax.gather(
                            v16, jnp.full((L, 1), r, jnp.int32), _GDN,
                            slice_sizes=(1,),
                            mode=lax.GatherScatterMode.PROMISE_IN_BOUNDS)
                        for k in range(DP // L):
                            seg = rows_v[b, c0 + r, pl.ds(k * L, L)]
                            rows_v[b, c0 + r, pl.ds(k * L, L)] = seg * v

                pltpu.async_copy(rows_v.at[b], acc.at[dst_v.at[j]], ssem[b],
                                 add=True)

                b2 = (b - 1) % NB
                j2 = j - 1
                jn = j2 + NB

                @pl.when(j2 >= 0)
                def _():
                    pltpu.make_async_copy(
                        rows_v.at[b2], acc.at[dst_v.at[j2]], ssem[b2]).wait()

                    @pl.when(jn < CPW)
                    def _():
                        pltpu.async_copy(
                            h_s.at[src_v.at[jn]], rows_v.at[b2], gsem[b2])
            return carry

        lax.fori_loop(0, CPW // NB, outer, 0)

        pltpu.make_async_copy(
            rows_v.at[NB - 1], acc.at[dst_v.at[CPW - 1]], ssem[NB - 1]).wait()

        plsc.subcore_barrier()

        for blk in range(NBLK):
            sl = pl.ds(base + blk * CHUNK, CHUNK)
            pltpu.sync_copy(acc.at[sl], rows_v.at[0])
            pltpu.sync_copy(rows_v.at[0], out_hbm.at[cid].at[sl])

    return pl.kernel(
        body,
        out_type=jax.ShapeDtypeStruct((NC, NP, DP), jnp.float32),
        mesh=plsc.VectorSubcoreMesh(core_axis_name="c", subcore_axis_name="s"),
        compiler_params=pltpu.CompilerParams(needs_layout_passes=False,
                                             use_tc_tiling_on_sc=False),
        scratch_types=[
            pltpu.VMEM((CPW, CHUNK), jnp.int32),
            pltpu.VMEM((CPW, CHUNK), jnp.int32),
            pltpu.VMEM((EPW,), jnp.float32),
            pltpu.VMEM((NB, CHUNK, DP), jnp.float32),
            pltpu.VMEM_SHARED((NP, DP), jnp.float32),
            pltpu.VMEM_SHARED((NP, DP), jnp.float32),
            [pltpu.SemaphoreType.DMA] * NB,
            [pltpu.SemaphoreType.DMA] * NB,
        ],
    )


_spmm1 = _make_spmm(1)
_spmm2 = _make_spmm(2)


def kernel(features, edge_index, edge_vals, W0, W1):
    w1p = jnp.pad(W1, ((0, 0), (0, DP - D_OUT)))
    h = _mlp(features, W0, w1p)

    pad = EP - E
    src = jnp.concatenate([edge_index[0], jnp.zeros((pad,), jnp.int32)])
    dst = jnp.concatenate([edge_index[1], jnp.zeros((pad,), jnp.int32)])
    vals = jnp.concatenate([edge_vals, jnp.zeros((pad,), jnp.float32)])
    src3 = src.reshape(NW, CPW, CHUNK)
    dst3 = dst.reshape(NW, CPW, CHUNK)
    vals2 = vals.reshape(NW, EPW)

    part = _spmm1(h, src3, dst3, vals2)
    part = _spmm2(part, src3, dst3, vals2)
    return _final(part)


# speedup vs baseline: 2.3455x; 1.1087x over previous
import jax
import jax.numpy as jnp
from jax import lax
from jax.experimental import pallas as pl
from jax.experimental.pallas import tpu as pltpu
from jax.experimental.pallas import tpu_sc as plsc

N = 10000
E = 320000
D_IN = 128
D_OUT = 40
HW = 20
NC, NS, L = 2, 16, 16
CHUNK = 128
NB = 4
CPT = 160
EPT = CPT * CHUNK
EP = EPT * NS
NP = 10240
RPT = NP // NS


def _mlp_body(x_ref, w0_ref, w1_ref, o_ref):
    h = jnp.maximum(
        jnp.dot(x_ref[...], w0_ref[...], preferred_element_type=jnp.float32), 0.0)
    h = jnp.dot(h, w1_ref[...], preferred_element_type=jnp.float32)
    o_ref[0, pl.ds(0, N), :] = h[:, :HW]
    o_ref[1, pl.ds(0, N), :] = h[:, HW:]


_mlp = pl.pallas_call(
    _mlp_body,
    out_shape=jax.ShapeDtypeStruct((NC, NP, HW), jnp.float32),
)


def _final_body(p_ref, o_ref):
    o_ref[...] = jnp.concatenate([p_ref[0, :N, :], p_ref[1, :N, :]], axis=1)


_final = pl.pallas_call(
    _final_body,
    out_shape=jax.ShapeDtypeStruct((N, D_OUT), jnp.float32),
)


_GDN = lax.GatherDimensionNumbers(
    offset_dims=(), collapsed_slice_dims=(0,), start_index_map=(0,))


def _spmm_body(parts_hbm, src_hbm, dst_hbm, vals_hbm, zeros_hbm, out_hbm,
               src_v, dst_v, vals_v, rows_v, stage_v, acc, h_s, gsem, ssem):
    cid = lax.axis_index("c")
    sid = lax.axis_index("s")
    sl = pl.ds(sid * RPT, RPT)

    iota = lax.iota(jnp.int32, L)
    quad = iota // 4
    qcol = L + iota - 4 * quad

    pltpu.sync_copy(zeros_hbm, stage_v)
    pltpu.sync_copy(stage_v, acc.at[sl])
    pltpu.sync_copy(parts_hbm.at[cid].at[sl], stage_v)
    pltpu.sync_copy(stage_v, h_s.at[sl])

    pltpu.sync_copy(src_hbm.at[sid], src_v)
    pltpu.sync_copy(dst_hbm.at[sid], dst_v)
    pltpu.sync_copy(vals_hbm.at[sid], vals_v)

    def spmm_round():
        plsc.subcore_barrier()

        for b in range(NB):
            pltpu.async_copy(h_s.at[src_v.at[b]], rows_v.at[b], gsem[b])

        def outer(g, carry):
            for b in range(NB):
                j = g * NB + b
                pltpu.make_async_copy(
                    h_s.at[src_v.at[j]], rows_v.at[b], gsem[b]).wait()

                @plsc.parallel_loop(0, CHUNK, L, unroll=2)
                def _scale(c0):
                    eb = j * CHUNK + c0
                    v16 = vals_v[pl.ds(eb, L)]
                    for r in range(L):
                        bc = lax.gather(
                            v16, jnp.full((L, 1), r, jnp.int32), _GDN,
                            slice_sizes=(1,),
                            mode=lax.GatherScatterMode.PROMISE_IN_BOUNDS)
                        seg = rows_v[b, c0 + r, pl.ds(0, L)]
                        rows_v[b, c0 + r, pl.ds(0, L)] = seg * bc
                    for t in range(4):
                        r_idx = jnp.full((L,), c0 + 4 * t, jnp.int32) + quad
                        b_idx = jnp.full((L,), b, jnp.int32)
                        vv = plsc.load_gather(
                            vals_v,
                            [jnp.full((L,), eb + 4 * t, jnp.int32) + quad])
                        seg = plsc.load_gather(rows_v, [b_idx, r_idx, qcol])
                        plsc.store_scatter(rows_v, [b_idx, r_idx, qcol],
                                           seg * vv)

                pltpu.async_copy(rows_v.at[b], acc.at[dst_v.at[j]], ssem[b],
                                 add=True)

                b2 = (b - 1) % NB
                j2 = j - 1
                jn = j2 + NB

                @pl.when(j2 >= 0)
                def _():
                    pltpu.make_async_copy(
                        rows_v.at[b2], acc.at[dst_v.at[j2]], ssem[b2]).wait()

                    @pl.when(jn < CPT)
                    def _():
                        pltpu.async_copy(
                            h_s.at[src_v.at[jn]], rows_v.at[b2], gsem[b2])
            return carry

        lax.fori_loop(0, CPT // NB, outer, 0)

        pltpu.make_async_copy(
            rows_v.at[NB - 1], acc.at[dst_v.at[CPT - 1]], ssem[NB - 1]).wait()

        plsc.subcore_barrier()

    spmm_round()

    pltpu.sync_copy(acc.at[sl], stage_v)
    pltpu.sync_copy(stage_v, h_s.at[sl])
    pltpu.sync_copy(zeros_hbm, stage_v)
    pltpu.sync_copy(stage_v, acc.at[sl])

    spmm_round()

    pltpu.sync_copy(acc.at[sl], stage_v)
    pltpu.sync_copy(stage_v, out_hbm.at[cid].at[sl])


_spmm = pl.kernel(
    _spmm_body,
    out_type=jax.ShapeDtypeStruct((NC, NP, HW), jnp.float32),
    mesh=plsc.VectorSubcoreMesh(core_axis_name="c", subcore_axis_name="s"),
    compiler_params=pltpu.CompilerParams(needs_layout_passes=False,
                                         use_tc_tiling_on_sc=False),
    scratch_types=[
        pltpu.VMEM((CPT, CHUNK), jnp.int32),
        pltpu.VMEM((CPT, CHUNK), jnp.int32),
        pltpu.VMEM((EPT,), jnp.float32),
        pltpu.VMEM((NB, CHUNK, HW), jnp.float32),
        pltpu.VMEM((RPT, HW), jnp.float32),
        pltpu.VMEM_SHARED((NP, HW), jnp.float32),
        pltpu.VMEM_SHARED((NP, HW), jnp.float32),
        [pltpu.SemaphoreType.DMA] * NB,
        [pltpu.SemaphoreType.DMA] * NB,
    ],
)


def kernel(features, edge_index, edge_vals, W0, W1):
    h = _mlp(features, W0, W1)

    pad = EP - E
    src = jnp.concatenate([edge_index[0], jnp.zeros((pad,), jnp.int32)])
    dst = jnp.concatenate([edge_index[1], jnp.zeros((pad,), jnp.int32)])
    vals = jnp.concatenate([edge_vals, jnp.zeros((pad,), jnp.float32)])
    src3 = src.reshape(NS, CPT, CHUNK)
    dst3 = dst.reshape(NS, CPT, CHUNK)
    vals2 = vals.reshape(NS, EPT)
    zeros = jnp.zeros((RPT, HW), jnp.float32)

    part = _spmm(h, src3, dst3, vals2, zeros)
    return _final(part)
